# baseline (speedup 1.0000x reference)
import functools

import jax
import jax.numpy as jnp
from jax import lax
from jax.experimental import pallas as pl
from jax.experimental.pallas import tpu as pltpu
from jax.experimental.pallas import tpu_sc as plsc

B, T = 16384, 200
D = 64
DP = 128
L = B * T
NC, NS = 2, 16
NW = NC * NS
PER_W = L // NW
G = 128
KPB = 8
BLK = KPB * G
NBLK = PER_W // BLK
NBUF = 2
VL = 16
RU = 4


def _make_kernel():
    mesh = plsc.VectorSubcoreMesh(core_axis_name="c", subcore_axis_name="s")

    @functools.partial(
        pl.kernel,
        mesh=mesh,
        out_type=jax.ShapeDtypeStruct((L, D), jnp.float32),
        scratch_types=[
            pltpu.VMEM((NBUF, KPB, G), jnp.int32),
            pltpu.VMEM((NBUF, G, DP), jnp.float32),
            pltpu.VMEM((NBUF, G, D), jnp.float32),
            pltpu.SemaphoreType.DMA((NBUF,)),
            pltpu.SemaphoreType.DMA((NBUF,)),
            pltpu.SemaphoreType.DMA((NBUF,)),
        ],
    )
    def k(idx_hbm, table_hbm, out_hbm, idx_v, rows_a, rows_b,
          sem_i, sem_g, sem_o):
        wid = lax.axis_index("s") * NC + lax.axis_index("c")
        base = wid * PER_W
        blk0 = wid * NBLK

        def idx_copy(blk, bb):
            return pltpu.make_async_copy(
                idx_hbm.at[blk0 + lax.min(blk, NBLK - 1)],
                idx_v.at[bb], sem_i.at[bb])

        def gather(bb, c, b):
            return pltpu.make_async_copy(
                table_hbm.at[idx_v.at[bb, c]],
                rows_a.at[b], sem_g.at[b])

        def out_copy(off, b):
            return pltpu.make_async_copy(
                rows_b.at[b], out_hbm.at[pl.ds(off, G)], sem_o.at[b])

        def repack(b):
            def rows(r, carry):
                for u in range(RU):
                    for v in range(D // VL):
                        s = pl.ds(v * VL, VL)
                        rows_b.at[b][r * RU + u, s] = rows_a.at[b][r * RU + u, s]
                return carry
            lax.fori_loop(0, G // RU, rows, 0)

        idx_copy(0, 0).start()

        def block_step(blk, bb):
            idx_copy(blk, bb).wait()
            idx_copy(blk + 1, 1 - bb).start()
            gather(bb, 0, 0).start()
            for c in range(KPB):
                b = c % NBUF
                g = blk * KPB + c
                gather(bb, c, b).wait()
                if c < KPB - 1:
                    gather(bb, c + 1, 1 - b).start()
                if c >= NBUF:
                    out_copy(base, b).wait()
                else:
                    @pl.when(blk >= 1)
                    def _():
                        out_copy(base, b).wait()
                repack(b)
                out_copy(base + g * G, b).start()

        def body(t, carry):
            for par in range(2):
                block_step(t * 2 + par, par)
            return carry

        lax.fori_loop(0, NBLK // 2, body, 0)

        for b in range(NBUF):
            out_copy(base, b).wait()
        idx_copy(NBLK - 1, 0).wait()

    return k


_gather_kernel = _make_kernel()


def kernel(indices, table):
    idx_3d = indices.astype(jnp.int32).reshape(L // BLK, KPB, G)
    table_p = jnp.pad(table, ((0, 0), (0, DP - D)))
    out = _gather_kernel(idx_3d, table_p)
    return out.reshape(B, T, D)
